# baseline (speedup 1.0000x reference)
import jax
import jax.numpy as jnp
from jax import lax
from jax.experimental import pallas as pl
from jax.experimental.pallas import tpu as pltpu
from jax.experimental.pallas import tpu_sc as plsc

NUM_PATCHES = 1024
D = 768
B = 64
NUM_WORKERS = 32
F_PER_W = D // NUM_WORKERS
LANES = 16
VECS_PER_ROW = D // LANES
IDX_PER_W = B * F_PER_W


NBUF = 4


def _emb_body(x_hbm, tok_hbm, pos_hbm, out_hbm,
              pos_v, idx_all, rows0, rows1, rows2, rows3,
              g0, g1, g2, g3, s0, s1, s2, s3):
    c = lax.axis_index("c")
    s = lax.axis_index("s")
    wid = s * 2 + c
    f0 = wid * F_PER_W

    rows = (rows0, rows1, rows2, rows3)
    gsem = (g0, g1, g2, g3)
    ssem = (s0, s1, s2, s3)

    pltpu.sync_copy(pos_hbm.at[pl.ds(f0, F_PER_W)], pos_v)
    pltpu.sync_copy(x_hbm.at[pl.ds(wid * IDX_PER_W, IDX_PER_W)], idx_all)

    def idx_slice(bb):
        return idx_all.at[pl.ds(bb * F_PER_W, F_PER_W)]

    def out_slice(bb):
        return out_hbm.at[pl.ds(bb * D + f0, F_PER_W)]

    for k in range(NBUF - 1):
        pltpu.async_copy(tok_hbm.at[idx_slice(k)], rows[k], gsem[k])

    def step(i, k):
        bb = NBUF * i + k
        cur = rows[k]
        prv = rows[(k + NBUF - 1) % NBUF]

        @pl.when(bb >= 1)
        def _():
            pltpu.make_async_copy(
                prv, out_slice(bb - 1), ssem[(k + NBUF - 1) % NBUF]).wait()

        @pl.when(bb + NBUF - 1 < B)
        def _():
            pltpu.async_copy(tok_hbm.at[idx_slice(bb + NBUF - 1)], prv,
                             gsem[(k + NBUF - 1) % NBUF])

        pltpu.make_async_copy(tok_hbm.at[idx_slice(bb)], cur, gsem[k]).wait()

        @pl.loop(0, F_PER_W)
        def _(r):
            for j in range(VECS_PER_ROW):
                sl = pl.ds(j * LANES, LANES)
                plsc.addupdate(cur.at[r, sl], pos_v[r, sl])

        pltpu.async_copy(cur, out_slice(bb), ssem[k])

    def body(i, carry):
        for k in range(NBUF):
            step(i, k)
        return carry

    lax.fori_loop(0, B // NBUF, body, 0)
    pltpu.make_async_copy(rows[(B - 1) % NBUF], out_slice(B - 1),
                          ssem[(B - 1) % NBUF]).wait()


@jax.jit
def kernel(x, token_table, pos_table):
    xp = x.reshape(B, NUM_WORKERS, F_PER_W).transpose(1, 0, 2).reshape(-1)
    mesh = plsc.VectorSubcoreMesh(core_axis_name="c", subcore_axis_name="s")
    out = pl.kernel(
        _emb_body,
        out_type=jax.ShapeDtypeStruct((B * D, D), jnp.float32),
        mesh=mesh,
        scratch_types=[
            pltpu.VMEM((F_PER_W, D), jnp.float32),
            pltpu.VMEM((IDX_PER_W,), jnp.int32),
            pltpu.VMEM((F_PER_W, D), jnp.float32),
            pltpu.VMEM((F_PER_W, D), jnp.float32),
            pltpu.VMEM((F_PER_W, D), jnp.float32),
            pltpu.VMEM((F_PER_W, D), jnp.float32),
            pltpu.SemaphoreType.DMA,
            pltpu.SemaphoreType.DMA,
            pltpu.SemaphoreType.DMA,
            pltpu.SemaphoreType.DMA,
            pltpu.SemaphoreType.DMA,
            pltpu.SemaphoreType.DMA,
            pltpu.SemaphoreType.DMA,
            pltpu.SemaphoreType.DMA,
        ],
    )(xp, token_table, pos_table)
    return out.reshape(B, D, D)
